# baseline (device time: 264159 ns/iter reference)
import jax
import jax.numpy as jnp
from jax import lax
from jax.experimental import pallas as pl
from jax.experimental.pallas import tpu as pltpu

N_DEV = 16
P = 4
Q = 4
M = 2048
N = 2048
QROWS = M // Q
N_S = 8
N_STREAMS = 2 * N_S
SLICE = QROWS // (2 * N_S)
STAGES = 8


def kernel(x, w_mat):
    def body(x_ref, w_ref, out_ref,
             acomm, bpre, bsuf,
             a_send, a_recv, b_send, b_recv, c_send, c_recv, c_credit):
        me = lax.axis_index("i")
        p = me // Q
        q = me % Q
        base = p * Q
        rgt = base + (q + 1) % Q
        lft = base + (q + Q - 1) % Q
        up = (me + Q) % N_DEV
        dn = (me + N_DEV - Q) % N_DEV

        def fwd_of(st):
            return st % 2 == 0

        def off_of(st):
            return (st // 2) if fwd_of(st) else N_S + st // 2

        def rows(jq, st):
            return pl.ds(jq * QROWS + off_of(st) * SLICE, SLICE)

        out_ref[pl.ds(q * QROWS, QROWS), :] = jnp.dot(
            x_ref[pl.ds(q * QROWS, QROWS), :], w_ref[:, :],
            preferred_element_type=jnp.float32,
        )

        @pl.when(p == P - 1)
        def _():
            for st in range(N_STREAMS):
                bsuf[st, :, :] = jnp.zeros((SLICE, N), jnp.float32)

        barrier_sem = pltpu.get_barrier_semaphore()
        for nbr in (lft, rgt, up, dn):
            pl.semaphore_signal(
                barrier_sem, inc=1,
                device_id=(nbr,), device_id_type=pl.DeviceIdType.MESH,
            )
        pl.semaphore_wait(barrier_sem, 4)

        a_rd = [[
            pltpu.make_async_remote_copy(
                src_ref=acomm.at[st, s],
                dst_ref=acomm.at[st, s + 1],
                send_sem=a_send.at[st, s],
                recv_sem=a_recv.at[st, s],
                device_id=(rgt if fwd_of(st) else lft,),
                device_id_type=pl.DeviceIdType.MESH,
            ) for s in range(3)]
            for st in range(N_STREAMS)]
        pre_rd = [
            pltpu.make_async_remote_copy(
                src_ref=acomm.at[st, 0], dst_ref=bpre.at[st],
                send_sem=b_send.at[st, 0], recv_sem=b_recv.at[st, 0],
                device_id=(up,), device_id_type=pl.DeviceIdType.MESH,
            )
            for st in range(N_STREAMS)
        ]
        suf_rd = [
            pltpu.make_async_remote_copy(
                src_ref=acomm.at[st, 1], dst_ref=bsuf.at[st],
                send_sem=b_send.at[st, 1], recv_sem=b_recv.at[st, 1],
                device_id=(dn,), device_id_type=pl.DeviceIdType.MESH,
            )
            for st in range(N_STREAMS)
        ]

        def c_quarter(st, a):
            if fwd_of(st):
                return (q + Q + 1 - a) % Q
            return (q + Q - 1 + a) % Q

        c_rd = [[
            pltpu.make_async_remote_copy(
                src_ref=out_ref.at[rows(c_quarter(st, a), st)],
                dst_ref=out_ref.at[rows(c_quarter(st, a), st)],
                send_sem=c_send.at[st, a],
                recv_sem=c_recv.at[st, a],
                device_id=(rgt if fwd_of(st) else lft,),
                device_id_type=pl.DeviceIdType.MESH,
            ) for a in range(3)]
            for st in range(N_STREAMS)]

        def stage(st, j):
            fwd = fwd_of(st)
            if j == 0:
                acomm[st, 0, :, :] = out_ref[rows(q, st), :]
                a_rd[st][0].start()
            elif j in (1, 2):
                a_rd[st][j - 1].wait_recv()
                cq = (q + Q - j) % Q if fwd else (q + j) % Q
                acomm[st, j, :, :] = (
                    acomm[st, j, :, :] + out_ref[rows(cq, st), :]
                )
                a_rd[st][j].start()
                if j == 2:
                    pl.semaphore_signal(
                        c_credit.at[st], inc=3,
                        device_id=(lft if fwd else rgt,),
                        device_id_type=pl.DeviceIdType.MESH,
                    )
            elif j == 3:
                a_rd[st][2].wait_recv()
                cq = (q + 1) % Q if fwd else (q + Q - 1) % Q
                acomm[st, 3, :, :] = (
                    acomm[st, 3, :, :] + out_ref[rows(cq, st), :]
                )
                a_rd[st][0].wait_send()
                a_rd[st][1].wait_send()

                @pl.when(p == 0)
                def _():
                    acomm[st, 0, :, :] = acomm[st, 3, :, :]
                    pre_rd[st].start()

                @pl.when(p == P - 1)
                def _():
                    acomm[st, 1, :, :] = acomm[st, 3, :, :]
                    suf_rd[st].start()
            elif j == 4:
                @pl.when(p > 0)
                def _():
                    pre_rd[st].wait_recv()
                    acomm[st, 0, :, :] = bpre[st, :, :] + acomm[st, 3, :, :]

                @pl.when((p > 0) & (p < P - 1))
                def _():
                    pre_rd[st].start()

                @pl.when(p < P - 1)
                def _():
                    suf_rd[st].wait_recv()
                    acomm[st, 1, :, :] = bsuf[st, :, :] + acomm[st, 3, :, :]

                @pl.when((p > 0) & (p < P - 1))
                def _():
                    suf_rd[st].start()

                own = (q + 1) % Q if fwd else (q + Q - 1) % Q
                out_ref[rows(own, st), :] = (
                    acomm[st, 0, :, :] + bsuf[st, :, :]
                )
                pl.semaphore_wait(c_credit.at[st], 1)
                c_rd[st][0].start()
            elif j in (5, 6):
                c_rd[st][j - 5].wait_recv()
                pl.semaphore_wait(c_credit.at[st], 1)
                c_rd[st][j - 4].start()
            elif j == 7:
                c_rd[st][2].wait_recv()

        for rnd in range(N_S + STAGES - 1):
            for st in range(N_STREAMS):
                j = rnd - st // 2
                if 0 <= j < STAGES:
                    stage(st, j)
            if rnd == 0:
                out_ref[:, :] = jnp.dot(
                    x_ref[:, :], w_ref[:, :],
                    preferred_element_type=jnp.float32,
                )

        for st in range(N_STREAMS):
            a_rd[st][2].wait_send()
            for s in range(3):
                c_rd[st][s].wait_send()

            @pl.when(p < P - 1)
            def _():
                pre_rd[st].wait_send()

            @pl.when(p > 0)
            def _():
                suf_rd[st].wait_send()

    return pl.pallas_call(
        body,
        out_shape=jax.ShapeDtypeStruct((M, N), jnp.float32),
        in_specs=[
            pl.BlockSpec(memory_space=pltpu.VMEM),
            pl.BlockSpec(memory_space=pltpu.VMEM),
        ],
        out_specs=pl.BlockSpec(memory_space=pltpu.VMEM),
        scratch_shapes=[
            pltpu.VMEM((N_STREAMS, 4, SLICE, N), jnp.float32),
            pltpu.VMEM((N_STREAMS, SLICE, N), jnp.float32),
            pltpu.VMEM((N_STREAMS, SLICE, N), jnp.float32),
            pltpu.SemaphoreType.DMA((N_STREAMS, 3)),
            pltpu.SemaphoreType.DMA((N_STREAMS, 3)),
            pltpu.SemaphoreType.DMA((N_STREAMS, 2)),
            pltpu.SemaphoreType.DMA((N_STREAMS, 2)),
            pltpu.SemaphoreType.DMA((N_STREAMS, 3)),
            pltpu.SemaphoreType.DMA((N_STREAMS, 3)),
            pltpu.SemaphoreType.REGULAR((N_STREAMS,)),
        ],
        compiler_params=pltpu.CompilerParams(collective_id=0),
    )(x, w_mat)


# device time: 181101 ns/iter; 1.4586x vs baseline; 1.4586x over previous
import jax
import jax.numpy as jnp
from jax import lax
from jax.experimental import pallas as pl
from jax.experimental.pallas import tpu as pltpu

N_DEV = 16
P = 4
Q = 4
M = 2048
N = 2048
QROWS = M // Q
N_S = 8
N_STREAMS = 2 * N_S
SLICE = QROWS // (2 * N_S)
STAGES = 9


def kernel(x, w_mat):
    def body(x_ref, w_ref, out_ref,
             acomm, bpre, bsuf,
             a_send, a_recv, b_send, b_recv, c_send, c_recv, c_credit):
        me = lax.axis_index("i")
        p = me // Q
        q = me % Q
        base = p * Q
        rgt = base + (q + 1) % Q
        lft = base + (q + Q - 1) % Q
        up = (me + Q) % N_DEV
        dn = (me + N_DEV - Q) % N_DEV

        def fwd_of(st):
            return st % 2 == 0

        def off_of(st):
            return (st // 2) if fwd_of(st) else N_S + st // 2

        def rows(jq, st):
            return pl.ds(jq * QROWS + off_of(st) * SLICE, SLICE)

        out_ref[pl.ds(q * QROWS, QROWS), :] = jnp.dot(
            x_ref[pl.ds(q * QROWS, QROWS), :], w_ref[:, :],
            preferred_element_type=jnp.float32,
        )

        @pl.when(p == P - 1)
        def _():
            for st in range(N_STREAMS):
                bsuf[st, :, :] = jnp.zeros((SLICE, N), jnp.float32)

        barrier_sem = pltpu.get_barrier_semaphore()
        for nbr in (lft, rgt, up, dn):
            pl.semaphore_signal(
                barrier_sem, inc=1,
                device_id=(nbr,), device_id_type=pl.DeviceIdType.MESH,
            )
        pl.semaphore_wait(barrier_sem, 4)

        a_rd = [[
            pltpu.make_async_remote_copy(
                src_ref=acomm.at[st, s],
                dst_ref=acomm.at[st, s + 1],
                send_sem=a_send.at[st, s],
                recv_sem=a_recv.at[st, s],
                device_id=(rgt if fwd_of(st) else lft,),
                device_id_type=pl.DeviceIdType.MESH,
            ) for s in range(3)]
            for st in range(N_STREAMS)]
        pre_rd = [
            pltpu.make_async_remote_copy(
                src_ref=acomm.at[st, 0], dst_ref=bpre.at[st],
                send_sem=b_send.at[st, 0], recv_sem=b_recv.at[st, 0],
                device_id=(up,), device_id_type=pl.DeviceIdType.MESH,
            )
            for st in range(N_STREAMS)
        ]
        suf_rd = [
            pltpu.make_async_remote_copy(
                src_ref=acomm.at[st, 1], dst_ref=bsuf.at[st],
                send_sem=b_send.at[st, 1], recv_sem=b_recv.at[st, 1],
                device_id=(dn,), device_id_type=pl.DeviceIdType.MESH,
            )
            for st in range(N_STREAMS)
        ]

        def c_quarter(st, a):
            if fwd_of(st):
                return (q + Q + 1 - a) % Q
            return (q + Q - 1 + a) % Q

        c_rd = [[
            pltpu.make_async_remote_copy(
                src_ref=out_ref.at[rows(c_quarter(st, a), st)],
                dst_ref=out_ref.at[rows(c_quarter(st, a), st)],
                send_sem=c_send.at[st, a],
                recv_sem=c_recv.at[st, a],
                device_id=(rgt if fwd_of(st) else lft,),
                device_id_type=pl.DeviceIdType.MESH,
            ) for a in range(3)]
            for st in range(N_STREAMS)]

        def stage(st, j):
            fwd = fwd_of(st)
            if j == 0:
                acomm[st, 0, :, :] = out_ref[rows(q, st), :]
                a_rd[st][0].start()
            elif j in (1, 2):
                a_rd[st][j - 1].wait_recv()
                cq = (q + Q - j) % Q if fwd else (q + j) % Q
                acomm[st, j, :, :] = (
                    acomm[st, j, :, :] + out_ref[rows(cq, st), :]
                )
                a_rd[st][j].start()
                if j == 2:
                    pl.semaphore_signal(
                        c_credit.at[st], inc=3,
                        device_id=(lft if fwd else rgt,),
                        device_id_type=pl.DeviceIdType.MESH,
                    )
            elif j == 3:
                a_rd[st][2].wait_recv()
                cq = (q + 1) % Q if fwd else (q + Q - 1) % Q
                acomm[st, 3, :, :] = (
                    acomm[st, 3, :, :] + out_ref[rows(cq, st), :]
                )
                a_rd[st][0].wait_send()
                a_rd[st][1].wait_send()

                @pl.when(p == 0)
                def _():
                    acomm[st, 0, :, :] = acomm[st, 3, :, :]
                    pre_rd[st].start()

                @pl.when(p == P - 1)
                def _():
                    acomm[st, 1, :, :] = acomm[st, 3, :, :]
                    suf_rd[st].start()
            elif j == 4:
                @pl.when(p > 0)
                def _():
                    pre_rd[st].wait_recv()
                    acomm[st, 0, :, :] = bpre[st, :, :] + acomm[st, 3, :, :]

                @pl.when((p > 0) & (p < P - 1))
                def _():
                    pre_rd[st].start()
            elif j == 5:
                @pl.when(p < P - 1)
                def _():
                    suf_rd[st].wait_recv()
                    acomm[st, 1, :, :] = bsuf[st, :, :] + acomm[st, 3, :, :]

                @pl.when((p > 0) & (p < P - 1))
                def _():
                    suf_rd[st].start()

                own = (q + 1) % Q if fwd else (q + Q - 1) % Q
                out_ref[rows(own, st), :] = (
                    acomm[st, 0, :, :] + bsuf[st, :, :]
                )
                pl.semaphore_wait(c_credit.at[st], 1)
                c_rd[st][0].start()
            elif j in (6, 7):
                c_rd[st][j - 6].wait_recv()
                pl.semaphore_wait(c_credit.at[st], 1)
                c_rd[st][j - 5].start()
            elif j == 8:
                c_rd[st][2].wait_recv()

        for rnd in range(N_S + STAGES - 1):
            for st in range(N_STREAMS):
                j = rnd - st // 2
                if 0 <= j < STAGES:
                    stage(st, j)
            if rnd == 0:
                out_ref[:, :] = jnp.dot(
                    x_ref[:, :], w_ref[:, :],
                    preferred_element_type=jnp.float32,
                )

        for st in range(N_STREAMS):
            a_rd[st][2].wait_send()
            for s in range(3):
                c_rd[st][s].wait_send()

            @pl.when(p < P - 1)
            def _():
                pre_rd[st].wait_send()

            @pl.when(p > 0)
            def _():
                suf_rd[st].wait_send()

    return pl.pallas_call(
        body,
        out_shape=jax.ShapeDtypeStruct((M, N), jnp.float32),
        in_specs=[
            pl.BlockSpec(memory_space=pltpu.VMEM),
            pl.BlockSpec(memory_space=pltpu.VMEM),
        ],
        out_specs=pl.BlockSpec(memory_space=pltpu.VMEM),
        scratch_shapes=[
            pltpu.VMEM((N_STREAMS, 4, SLICE, N), jnp.float32),
            pltpu.VMEM((N_STREAMS, SLICE, N), jnp.float32),
            pltpu.VMEM((N_STREAMS, SLICE, N), jnp.float32),
            pltpu.SemaphoreType.DMA((N_STREAMS, 3)),
            pltpu.SemaphoreType.DMA((N_STREAMS, 3)),
            pltpu.SemaphoreType.DMA((N_STREAMS, 2)),
            pltpu.SemaphoreType.DMA((N_STREAMS, 2)),
            pltpu.SemaphoreType.DMA((N_STREAMS, 3)),
            pltpu.SemaphoreType.DMA((N_STREAMS, 3)),
            pltpu.SemaphoreType.REGULAR((N_STREAMS,)),
        ],
        compiler_params=pltpu.CompilerParams(collective_id=0),
    )(x, w_mat)
